# baseline (speedup 1.0000x reference)
import functools

import jax
import jax.numpy as jnp
import numpy as np
from jax.experimental import pallas as pl
from jax.experimental.pallas import tpu as pltpu
from jax.experimental.pallas import tpu_sc as plsc

_NUM_BUCKETS = 32
_N_HEADS = 16
_MAX_DISTANCE = 128
_SEQ = 2048
_VT_W = 4352
_DMA_LAG = 64


def _vtab_body(emb_smem, out_ref):
    s = jax.lax.broadcasted_iota(jnp.int32, (8, _VT_W), 0)
    x = jax.lax.broadcasted_iota(jnp.int32, (8, _VT_W), 1)
    d = x + s - (_SEQ - 1)
    n = -d
    side = jnp.where(n < 0, _NUM_BUCKETS // 2, 0)
    na = jnp.abs(n)
    max_exact = _NUM_BUCKETS // 4
    is_small = na < max_exact
    nf = jnp.maximum(na, 1).astype(jnp.float32)
    val_large = max_exact + (
        jnp.log(nf / max_exact)
        / np.log(_MAX_DISTANCE / max_exact)
        * (_NUM_BUCKETS // 2 - max_exact)
    ).astype(jnp.int32)
    val_large = jnp.minimum(val_large, _NUM_BUCKETS // 2 - 1)
    bucket = side + jnp.where(is_small, na, val_large)
    for h in range(_N_HEADS):
        v = jnp.full((8, _VT_W), emb_smem[0, h], jnp.float32)
        for b in range(1, _NUM_BUCKETS):
            v = jnp.where(bucket == b, emb_smem[b, h], v)
        out_ref[h] = v


def _build_vtab(embeddings):
    return pl.pallas_call(
        _vtab_body,
        out_shape=jax.ShapeDtypeStruct((_N_HEADS, 8, _VT_W), jnp.float32),
        in_specs=[pl.BlockSpec(memory_space=pltpu.SMEM)],
    )(embeddings)


def _expand_body(vtab_hbm, out_hbm, vt, sem):
    c = jax.lax.axis_index("c")
    s = jax.lax.axis_index("s")
    wid = s * 2 + c
    h = wid // 2
    half = wid % 2
    for s8 in range(8):
        pltpu.sync_copy(
            vtab_hbm.at[h, s8], vt.at[pl.ds(s8 * _VT_W, _VT_W)]
        )
    i0 = half * (_SEQ // 2)
    rows_base = h * _SEQ + i0

    def fire(r, carry):
        i = i0 + r
        start = (_SEQ - 1) - i
        s8 = jnp.bitwise_and(start, 7)
        src_off = pl.multiple_of(s8 * _VT_W + (start - s8), 8)
        dst_off = pl.multiple_of((rows_base + r) * _SEQ, 8)
        pltpu.async_copy(
            vt.at[pl.ds(src_off, _SEQ)], out_hbm.at[pl.ds(dst_off, _SEQ)], sem
        )

        @pl.when(r >= _DMA_LAG)
        def _():
            pltpu.make_async_copy(
                vt.at[pl.ds(0, _SEQ)], out_hbm.at[pl.ds(0, _SEQ)], sem
            ).wait()

        return carry

    jax.lax.fori_loop(0, _SEQ // 2, fire, 0, unroll=False)

    def drain(r, carry):
        pltpu.make_async_copy(
            vt.at[pl.ds(0, _SEQ)], out_hbm.at[pl.ds(0, _SEQ)], sem
        ).wait()
        return carry

    jax.lax.fori_loop(0, _DMA_LAG, drain, 0, unroll=False)


@functools.lru_cache(maxsize=1)
def _make_expand():
    return pl.kernel(
        _expand_body,
        out_type=jax.ShapeDtypeStruct((_N_HEADS * _SEQ * _SEQ,), jnp.float32),
        mesh=plsc.VectorSubcoreMesh(core_axis_name="c", subcore_axis_name="s"),
        scratch_types=[
            pltpu.VMEM((8 * _VT_W,), jnp.float32),
            pltpu.SemaphoreType.DMA,
        ],
    )


def kernel(q, k, embeddings):
    vtab8 = _build_vtab(embeddings)
    out = _make_expand()(vtab8)
    return out.reshape(1, _N_HEADS, _SEQ, _SEQ)


# speedup vs baseline: 1.0105x; 1.0105x over previous
import functools

import jax
import jax.numpy as jnp
import numpy as np
from jax.experimental import pallas as pl
from jax.experimental.pallas import tpu as pltpu
from jax.experimental.pallas import tpu_sc as plsc

_NUM_BUCKETS = 32
_N_HEADS = 16
_MAX_DISTANCE = 128
_SEQ = 2048
_VT_W = 4352
_DMA_LAG = 64


def _vtab_body(emb_smem, out_ref):
    s = jax.lax.broadcasted_iota(jnp.int32, (8, _VT_W), 0)
    x = jax.lax.broadcasted_iota(jnp.int32, (8, _VT_W), 1)
    d = x + s - (_SEQ - 1)
    n = -d
    side = jnp.where(n < 0, _NUM_BUCKETS // 2, 0)
    na = jnp.abs(n)
    max_exact = _NUM_BUCKETS // 4
    is_small = na < max_exact
    nf = jnp.maximum(na, 1).astype(jnp.float32)
    val_large = max_exact + (
        jnp.log(nf / max_exact)
        / np.log(_MAX_DISTANCE / max_exact)
        * (_NUM_BUCKETS // 2 - max_exact)
    ).astype(jnp.int32)
    val_large = jnp.minimum(val_large, _NUM_BUCKETS // 2 - 1)
    bucket = side + jnp.where(is_small, na, val_large)
    for h in range(_N_HEADS):
        v = jnp.full((8, _VT_W), emb_smem[0, h], jnp.float32)
        for b in range(1, _NUM_BUCKETS):
            v = jnp.where(bucket == b, emb_smem[b, h], v)
        out_ref[h] = v


def _build_vtab(embeddings):
    return pl.pallas_call(
        _vtab_body,
        out_shape=jax.ShapeDtypeStruct((_N_HEADS, 8, _VT_W), jnp.float32),
        in_specs=[pl.BlockSpec(memory_space=pltpu.SMEM)],
    )(embeddings)


def _expand_body(vtab_hbm, out_hbm, vt, sem):
    c = jax.lax.axis_index("c")
    s = jax.lax.axis_index("s")
    wid = s * 2 + c
    h = wid // 2
    half = wid % 2
    stage = [
        pltpu.async_copy(
            vtab_hbm.at[h, s8], vt.at[pl.ds(s8 * _VT_W, _VT_W)], sem
        )
        for s8 in range(8)
    ]
    for d in stage:
        d.wait()
    i0 = half * (_SEQ // 2)
    rows_base = h * _SEQ + i0

    def fire(r, carry):
        i = i0 + r
        start = (_SEQ - 1) - i
        s8 = jnp.bitwise_and(start, 7)
        src_off = pl.multiple_of(s8 * _VT_W + (start - s8), 8)
        dst_off = pl.multiple_of((rows_base + r) * _SEQ, 8)
        pltpu.async_copy(
            vt.at[pl.ds(src_off, _SEQ)], out_hbm.at[pl.ds(dst_off, _SEQ)], sem
        )

        @pl.when(r >= _DMA_LAG)
        def _():
            pltpu.make_async_copy(
                vt.at[pl.ds(0, _SEQ)], out_hbm.at[pl.ds(0, _SEQ)], sem
            ).wait()

        return carry

    jax.lax.fori_loop(0, _SEQ // 2, fire, 0, unroll=False)

    def drain(r, carry):
        pltpu.make_async_copy(
            vt.at[pl.ds(0, _SEQ)], out_hbm.at[pl.ds(0, _SEQ)], sem
        ).wait()
        return carry

    jax.lax.fori_loop(0, _DMA_LAG, drain, 0, unroll=False)


@functools.lru_cache(maxsize=1)
def _make_expand():
    return pl.kernel(
        _expand_body,
        out_type=jax.ShapeDtypeStruct((_N_HEADS * _SEQ * _SEQ,), jnp.float32),
        mesh=plsc.VectorSubcoreMesh(core_axis_name="c", subcore_axis_name="s"),
        scratch_types=[
            pltpu.VMEM((8 * _VT_W,), jnp.float32),
            pltpu.SemaphoreType.DMA,
        ],
    )


def kernel(q, k, embeddings):
    vtab8 = _build_vtab(embeddings)
    out = _make_expand()(vtab8)
    return out.reshape(1, _N_HEADS, _SEQ, _SEQ)


# speedup vs baseline: 1.0226x; 1.0120x over previous
import functools

import jax
import jax.numpy as jnp
import numpy as np
from jax.experimental import pallas as pl
from jax.experimental.pallas import tpu as pltpu
from jax.experimental.pallas import tpu_sc as plsc

_NUM_BUCKETS = 32
_N_HEADS = 16
_MAX_DISTANCE = 128
_SEQ = 2048
_VT_W = 4352
_DMA_LAG = 64


_WIN_LO = 1920
_WIN_HI = 2176


def _vtab_body(emb_smem, out_ref):
    win = _WIN_HI - _WIN_LO
    s = jax.lax.broadcasted_iota(jnp.int32, (8, win), 0)
    x = jax.lax.broadcasted_iota(jnp.int32, (8, win), 1) + _WIN_LO
    d = x + s - (_SEQ - 1)
    n = -d
    side = jnp.where(n < 0, _NUM_BUCKETS // 2, 0)
    na = jnp.abs(n)
    max_exact = _NUM_BUCKETS // 4
    is_small = na < max_exact
    nf = jnp.maximum(na, 1).astype(jnp.float32)
    val_large = max_exact + (
        jnp.log(nf / max_exact)
        / np.log(_MAX_DISTANCE / max_exact)
        * (_NUM_BUCKETS // 2 - max_exact)
    ).astype(jnp.int32)
    val_large = jnp.minimum(val_large, _NUM_BUCKETS // 2 - 1)
    bucket = side + jnp.where(is_small, na, val_large)
    for h in range(_N_HEADS):
        v = jnp.full((8, win), emb_smem[0, h], jnp.float32)
        for b in range(1, _NUM_BUCKETS):
            v = jnp.where(bucket == b, emb_smem[b, h], v)
        out_ref[h, :, :_WIN_LO] = jnp.full(
            (8, _WIN_LO), emb_smem[_NUM_BUCKETS // 2 - 1, h], jnp.float32
        )
        out_ref[h, :, _WIN_LO:_WIN_HI] = v
        out_ref[h, :, _WIN_HI:] = jnp.full(
            (8, _VT_W - _WIN_HI), emb_smem[_NUM_BUCKETS - 1, h], jnp.float32
        )


def _build_vtab(embeddings):
    return pl.pallas_call(
        _vtab_body,
        out_shape=jax.ShapeDtypeStruct((_N_HEADS, 8, _VT_W), jnp.float32),
        in_specs=[pl.BlockSpec(memory_space=pltpu.SMEM)],
    )(embeddings)


def _expand_body(vtab_hbm, out_hbm, vt, sem):
    c = jax.lax.axis_index("c")
    s = jax.lax.axis_index("s")
    wid = s * 2 + c
    h = wid // 2
    half = wid % 2
    stage = [
        pltpu.async_copy(
            vtab_hbm.at[h, s8], vt.at[pl.ds(s8 * _VT_W, _VT_W)], sem
        )
        for s8 in range(8)
    ]
    for d in stage:
        d.wait()
    i0 = half * (_SEQ // 2)
    rows_base = h * _SEQ + i0

    def fire_one(r):
        i = i0 + r
        start = (_SEQ - 1) - i
        s8 = jnp.bitwise_and(start, 7)
        src_off = pl.multiple_of(s8 * _VT_W + (start - s8), 8)
        dst_off = pl.multiple_of((rows_base + r) * _SEQ, 8)
        pltpu.async_copy(
            vt.at[pl.ds(src_off, _SEQ)], out_hbm.at[pl.ds(dst_off, _SEQ)], sem
        )

    def wait_one():
        pltpu.make_async_copy(
            vt.at[pl.ds(0, _SEQ)], out_hbm.at[pl.ds(0, _SEQ)], sem
        ).wait()

    def prologue(r, carry):
        fire_one(r)
        return carry

    jax.lax.fori_loop(0, _DMA_LAG, prologue, 0, unroll=False)

    def steady(r, carry):
        fire_one(r)
        wait_one()
        return carry

    jax.lax.fori_loop(_DMA_LAG, _SEQ // 2, steady, 0, unroll=False)

    def drain(r, carry):
        wait_one()
        return carry

    jax.lax.fori_loop(0, _DMA_LAG, drain, 0, unroll=False)


@functools.lru_cache(maxsize=1)
def _make_expand():
    return pl.kernel(
        _expand_body,
        out_type=jax.ShapeDtypeStruct((_N_HEADS * _SEQ * _SEQ,), jnp.float32),
        mesh=plsc.VectorSubcoreMesh(core_axis_name="c", subcore_axis_name="s"),
        scratch_types=[
            pltpu.VMEM((8 * _VT_W,), jnp.float32),
            pltpu.SemaphoreType.DMA,
        ],
    )


def kernel(q, k, embeddings):
    vtab8 = _build_vtab(embeddings)
    out = _make_expand()(vtab8)
    return out.reshape(1, _N_HEADS, _SEQ, _SEQ)
